# baseline (speedup 1.0000x reference)
import jax
import jax.numpy as jnp
from jax.experimental import pallas as pl
from jax.experimental.pallas import tpu as pltpu

_CHUNKS0 = 4


def _dot_nt(a, b):
    return jax.lax.dot_general(
        a, b, (((1,), (1,)), ((), ())), preferred_element_type=jnp.float32
    )


def _gnn_fwd(
    data_hbm, graph_hbm,
    fw0, fb0, f1w0, f1b0,
    fw1, fb1, f1w1, f1b1,
    fw2, fb2, f1w2, f1b2,
    gw, gb, cw0, cb0, cw1, cb1,
    out_ref,
    d0, d1, d2, graph_vmem, mm_scr, sems, sg0, sg1,
):
    N = graph_vmem.shape[0]
    ck = N // _CHUNKS0
    half = N // 2

    cps0 = [
        pltpu.make_async_copy(
            data_hbm.at[0, pl.ds(c * ck, ck)], d0.at[pl.ds(c * ck, ck)],
            sems.at[c],
        )
        for c in range(_CHUNKS0)
    ]
    cp1 = pltpu.make_async_copy(data_hbm.at[1], d1, sems.at[_CHUNKS0])
    cp2 = pltpu.make_async_copy(data_hbm.at[2], d2, sems.at[_CHUNKS0 + 1])
    cpg0 = pltpu.make_async_copy(
        graph_hbm.at[pl.ds(0, half)], graph_vmem.at[pl.ds(0, half)], sg0
    )
    cpg1 = pltpu.make_async_copy(
        graph_hbm.at[pl.ds(half, half)], graph_vmem.at[pl.ds(half, half)], sg1
    )
    for cp in cps0:
        cp.start()
    cp1.start()
    cp2.start()
    cpg0.start()
    cpg1.start()

    for c in range(_CHUNKS0):
        cps0[c].wait()
        rows = pl.ds(c * ck, ck)
        h = jnp.maximum(_dot_nt(d0[rows, :], fw0[...]) + fb0[...], 0.0)
        h = jnp.maximum(_dot_nt(h, f1w0[...]) + f1b0[...], 0.0)
        mm_scr[rows, pl.ds(0, 128)] = h

    cp1.wait()
    h = jnp.maximum(_dot_nt(d1[...], fw1[...]) + fb1[...], 0.0)
    mm_scr[:, pl.ds(128, 128)] = jnp.maximum(_dot_nt(h, f1w1[...]) + f1b1[...], 0.0)
    cp2.wait()
    h = jnp.maximum(_dot_nt(d2[...], fw2[...]) + fb2[...], 0.0)
    mm_scr[:, pl.ds(256, 128)] = jnp.maximum(_dot_nt(h, f1w2[...]) + f1b2[...], 0.0)

    mm = mm_scr[...]
    xw = _dot_nt(mm, gw[...])

    cpg0.wait()
    colsum0 = jnp.sum(graph_vmem[pl.ds(0, half), :], axis=0)
    cpg1.wait()
    deg = colsum0 + jnp.sum(graph_vmem[pl.ds(half, half), :], axis=0) + 1.0
    dis = jnp.where(deg > 0, jax.lax.rsqrt(jnp.maximum(deg, 1e-12)), 0.0)
    sx = xw * dis[:, None]
    y = jax.lax.dot_general(
        graph_vmem[...], sx, (((0,), (0,)), ((), ())),
        preferred_element_type=jnp.float32,
    )
    g = dis[:, None] * (y + sx) + gb[...]

    z = jnp.concatenate([mm, g], axis=1)
    h = _dot_nt(z, cw0[...]) + cb0[...]
    h = jnp.where(h >= 0, h, 0.01 * h)
    out_ref[...] = _dot_nt(h, cw1[...]) + cb1[...]


def kernel(data_list, graph, fc_w0, fc_b0, fc1_w0, fc1_b0, fc_w1, fc_b1,
           fc1_w1, fc1_b1, fc_w2, fc_b2, fc1_w2, fc1_b2, gcn_w, gcn_b,
           cls_w0, cls_b0, cls_w1, cls_b1):
    V, N, D = data_list.shape
    H0 = gcn_b.shape[0]
    C = cls_w1.shape[0]
    vmem = pl.BlockSpec(memory_space=pltpu.VMEM)
    return pl.pallas_call(
        _gnn_fwd,
        in_specs=[
            pl.BlockSpec(memory_space=pl.ANY),
            pl.BlockSpec(memory_space=pl.ANY),
        ] + [vmem] * 18,
        out_specs=pl.BlockSpec(memory_space=pltpu.VMEM),
        out_shape=jax.ShapeDtypeStruct((N, C), jnp.float32),
        scratch_shapes=[
            pltpu.VMEM((N, D), jnp.float32),
            pltpu.VMEM((N, D), jnp.float32),
            pltpu.VMEM((N, D), jnp.float32),
            pltpu.VMEM((N, N), jnp.float32),
            pltpu.VMEM((N, 3 * H0), jnp.float32),
            pltpu.SemaphoreType.DMA((_CHUNKS0 + 2,)),
            pltpu.SemaphoreType.DMA,
            pltpu.SemaphoreType.DMA,
        ],
    )(data_list, graph, fc_w0, fc_b0, fc1_w0, fc1_b0, fc_w1, fc_b1,
      fc1_w1, fc1_b1, fc_w2, fc_b2, fc1_w2, fc1_b2, gcn_w, gcn_b,
      cls_w0, cls_b0, cls_w1, cls_b1)
